# Initial kernel scaffold; baseline (speedup 1.0000x reference)
import jax
import jax.numpy as jnp
from jax import lax
from jax.experimental import pallas as pl
from jax.experimental.pallas import tpu as pltpu
from jax.experimental.pallas import tpu_sc as plsc

NN = 10000
EE = 320000
HH = 8
CC = 64
HC = 512
N2 = 10240
NC = 2
NS = 16
NW = NC * NS


def _tc1_body(x_ref, wg_ref, ps_ref, pd_ref, h_ref, as_ref, ad_ref):
    xb = x_ref[...]
    hb = jnp.dot(xb, wg_ref[...], preferred_element_type=jnp.float32)
    h_ref[...] = hb
    as_ref[...] = jnp.dot(hb, ps_ref[...], preferred_element_type=jnp.float32)
    ad_ref[...] = jnp.dot(hb, pd_ref[...], preferred_element_type=jnp.float32)


def _tc1(x, W_gat, PS, PD):
    R = 1000
    return pl.pallas_call(
        _tc1_body,
        grid=(NN // R,),
        in_specs=[pl.BlockSpec((R, 128), lambda i: (i, 0)),
                  pl.BlockSpec((128, HC), lambda i: (0, 0)),
                  pl.BlockSpec((HC, 16), lambda i: (0, 0)),
                  pl.BlockSpec((HC, 16), lambda i: (0, 0))],
        out_specs=[pl.BlockSpec((R, HC), lambda i: (i, 0)),
                   pl.BlockSpec((R, 16), lambda i: (i, 0)),
                   pl.BlockSpec((R, 16), lambda i: (i, 0))],
        out_shape=[jax.ShapeDtypeStruct((NN, HC), jnp.float32),
                   jax.ShapeDtypeStruct((NN, 16), jnp.float32),
                   jax.ShapeDtypeStruct((NN, 16), jnp.float32)],
    )(x, W_gat, PS, PD)


def _tc2_body(x1_ref, wl_ref, wr_ref, bs_ref, y1_ref, z_ref):
    xb = x1_ref[...]
    y1_ref[...] = jnp.dot(xb, wl_ref[...], preferred_element_type=jnp.float32)
    z_ref[...] = jnp.dot(xb, wr_ref[...], preferred_element_type=jnp.float32) + bs_ref[...]


def _tc2(x1, Wl, Wr, bs):
    R = 512
    return pl.pallas_call(
        _tc2_body,
        grid=(N2 // R,),
        in_specs=[pl.BlockSpec((R, HC), lambda i: (i, 0)),
                  pl.BlockSpec((HC, CC), lambda i: (0, 0)),
                  pl.BlockSpec((HC, CC), lambda i: (0, 0)),
                  pl.BlockSpec((1, CC), lambda i: (0, 0))],
        out_specs=[pl.BlockSpec((R, CC), lambda i: (i, 0)),
                   pl.BlockSpec((R, CC), lambda i: (i, 0))],
        out_shape=[jax.ShapeDtypeStruct((N2, CC), jnp.float32),
                   jax.ShapeDtypeStruct((N2, CC), jnp.float32)],
    )(x1, Wl, Wr, bs)


def _tc3_body(x2_ref, w1_ref, b1_ref, w2_ref, b2_ref, o_ref):
    xb = x2_ref[...]
    t = jnp.maximum(
        jnp.dot(xb, w1_ref[...], preferred_element_type=jnp.float32) + b1_ref[...], 0.0)
    o_ref[...] = jnp.dot(t, w2_ref[...], preferred_element_type=jnp.float32) + b2_ref[...]


def _tc3(x2, W1, b1, W2, b2):
    R = 512
    return pl.pallas_call(
        _tc3_body,
        grid=(N2 // R,),
        in_specs=[pl.BlockSpec((R, CC), lambda i: (i, 0)),
                  pl.BlockSpec((CC, 32), lambda i: (0, 0)),
                  pl.BlockSpec((1, 32), lambda i: (0, 0)),
                  pl.BlockSpec((32, 1), lambda i: (0, 0)),
                  pl.BlockSpec((1, 1), lambda i: (0, 0))],
        out_specs=pl.BlockSpec((R, 1), lambda i: (i, 0)),
        out_shape=jax.ShapeDtypeStruct((N2, 1), jnp.float32),
    )(x2, W1, b1, W2, b2)


_B1B = 400
_B1NB = EE // NW // _B1B


def _b1_kernel(src_h, dst_h, ew_h, as_h, ad_h, crep_h, rec_h, dpart_h,
               srcb, dstb, ewb, rowS, rowD, stg, crepv, shd, semS, semD):
    cid = lax.axis_index("c")
    sid = lax.axis_index("s")
    wid = sid * NC + cid
    lane = lax.iota(jnp.int32, 16)
    pltpu.sync_copy(crep_h, crepv)
    crep = crepv[...]

    @pl.loop(0, _B1B)
    def _zs(i):
        stg[i] = jnp.zeros((16,), jnp.float32)
    pltpu.sync_copy(stg, shd.at[pl.ds(sid * 640, _B1B)])
    pltpu.sync_copy(stg.at[pl.ds(0, 640 - _B1B)],
                    shd.at[pl.ds(sid * 640 + _B1B, 640 - _B1B)])
    plsc.subcore_barrier()

    ebase = wid * (EE // NW)

    @pl.loop(0, _B1NB)
    def _batch(b):
        base = ebase + b * _B1B
        pltpu.sync_copy(src_h.at[pl.ds(base, _B1B)], srcb)
        pltpu.sync_copy(dst_h.at[pl.ds(base, _B1B)], dstb)
        pltpu.sync_copy(ew_h.at[pl.ds(base, _B1B)], ewb)
        cpS = pltpu.async_copy(as_h.at[srcb], rowS, semS)
        cpD = pltpu.async_copy(ad_h.at[dstb], rowD, semD)
        cpS.wait()
        cpD.wait()

        @pl.loop(0, _B1B)
        def _e(i):
            iv = jnp.full((16,), i, jnp.int32)
            s = rowS[i]
            d = rowD[i]
            ewv = plsc.load_gather(ewb, [iv])
            lg = s + d + ewv * crep
            ex = jnp.exp(jnp.maximum(lg, 0.2 * lg))
            srcf = plsc.bitcast(plsc.load_gather(srcb, [iv]), jnp.float32)
            val = jnp.where(lane < 8, ex,
                            jnp.where(lane == 8, jnp.ones((16,), jnp.float32),
                                      jnp.where(lane == 9, srcf,
                                                jnp.zeros((16,), jnp.float32))))
            stg[i] = val

        pltpu.sync_copy(stg, rec_h.at[pl.ds(base, _B1B)])
        pltpu.sync_copy(stg, shd.at[dstb], add=True)

    plsc.subcore_barrier()
    pltpu.sync_copy(shd.at[pl.ds(sid * 640, 640)],
                    dpart_h.at[cid, pl.ds(sid * 640, 640)])


def _b1(src, dst, ew, As, Ad, crep):
    mesh = plsc.VectorSubcoreMesh(core_axis_name="c", subcore_axis_name="s",
                                  num_cores=NC, num_subcores=NS)
    f = pl.kernel(
        _b1_kernel,
        out_type=[jax.ShapeDtypeStruct((EE, 16), jnp.float32),
                  jax.ShapeDtypeStruct((NC, N2, 16), jnp.float32)],
        mesh=mesh,
        scratch_types=[
            pltpu.VMEM((_B1B,), jnp.int32),
            pltpu.VMEM((_B1B,), jnp.int32),
            pltpu.VMEM((_B1B,), jnp.float32),
            pltpu.VMEM((_B1B, 16), jnp.float32),
            pltpu.VMEM((_B1B, 16), jnp.float32),
            pltpu.VMEM((_B1B, 16), jnp.float32),
            pltpu.VMEM((16,), jnp.float32),
            pltpu.VMEM_SHARED((N2, 16), jnp.float32),
            pltpu.SemaphoreType.DMA,
            pltpu.SemaphoreType.DMA,
        ])
    return f(src, dst, ew, As, Ad, crep)


_CW = 4000
_CG = 32
_CCH = 160


def _c_kernel(dst_h, rec_h, h_hbm, dpart_h, bg_h, x1_h,
              dstw, mids, mdst, recb, srcg, hbuf, acc, dloc, dloc2, bgv, sem):
    cid = lax.axis_index("c")
    sid = lax.axis_index("s")
    wid = sid * NC + cid
    lane = lax.iota(jnp.int32, 16)
    pltpu.sync_copy(bg_h, bgv)

    @pl.loop(0, 251)
    def _zm(k):
        mids[pl.ds(k * 16, 16)] = jnp.zeros((16,), jnp.int32)

    @pl.loop(0, 2)
    def _round(r):
        chunk = r * NW + wid
        lo = chunk * _CCH

        @pl.loop(0, _CCH)
        def _za(i):
            @pl.loop(0, HC // 16)
            def _zv(v):
                acc[i, pl.ds(v * 16, 16)] = jnp.zeros((16,), jnp.float32)

        @pl.loop(0, EE // _CW)
        def _win(w):
            wb = w * _CW
            pltpu.sync_copy(dst_h.at[pl.ds(wb, _CW)], dstw)

            def _scan(k, cnt):
                dl = dstw[pl.ds(k * 16, 16)] - lo
                msk = (dl >= 0) & (dl < _CCH)
                ids = lane + (wb + k * 16)
                plsc.store_compressed(mids.at[pl.ds(cnt, 16)], ids, mask=msk)
                plsc.store_compressed(mdst.at[pl.ds(cnt, 16)], dl, mask=msk)
                return cnt + jnp.sum(msk.astype(jnp.int32))

            cnt = lax.fori_loop(0, _CW // 16, _scan, 0)
            nb = (cnt + _CG - 1) // _CG

            @pl.loop(0, nb)
            def _bat(b):
                mb = b * _CG
                pltpu.async_copy(rec_h.at[mids.at[pl.ds(mb, _CG)]], recb, sem).wait()

                @pl.loop(0, _CG // 16)
                def _sx(k2):
                    rows = lane + k2 * 16
                    cols = jnp.full((16,), 9, jnp.int32)
                    sv = plsc.bitcast(plsc.load_gather(recb, [rows, cols]), jnp.int32)
                    srcg[pl.ds(k2 * 16, 16)] = sv

                pltpu.async_copy(h_hbm.at[srcg], hbuf, sem).wait()

                @pl.loop(0, _CG)
                def _acc(g):
                    @pl.when(mb + g < cnt)
                    def _():
                        row = mdst[mb + g]
                        for h8 in range(HH):
                            ab = plsc.load_gather(
                                recb, [jnp.full((16,), g, jnp.int32),
                                       jnp.full((16,), h8, jnp.int32)])
                            for v in range(4):
                                col = h8 * CC + v * 16
                                hv = hbuf[g, pl.ds(col, 16)]
                                plsc.addupdate(acc.at[row, pl.ds(col, 16)], hv * ab)

        pltpu.sync_copy(dpart_h.at[0, pl.ds(lo, _CCH)], dloc)
        pltpu.sync_copy(dpart_h.at[1, pl.ds(lo, _CCH)], dloc2)

        @pl.loop(0, _CCH)
        def _fin(i):
            dloc[i] = dloc[i] + dloc2[i] + jnp.full((16,), 1e-16, jnp.float32)
            for h8 in range(HH):
                den = plsc.load_gather(
                    dloc, [jnp.full((16,), i, jnp.int32),
                           jnp.full((16,), h8, jnp.int32)])
                for v in range(4):
                    col = h8 * CC + v * 16
                    acc[i, pl.ds(col, 16)] = jnp.maximum(
                        acc[i, pl.ds(col, 16)] / den + bgv[pl.ds(col, 16)], 0.0)

        pltpu.sync_copy(acc, x1_h.at[pl.ds(lo, _CCH)])


def _c(dst, rec, h, dpart, bg):
    mesh = plsc.VectorSubcoreMesh(core_axis_name="c", subcore_axis_name="s",
                                  num_cores=NC, num_subcores=NS)
    f = pl.kernel(
        _c_kernel,
        out_type=[jax.ShapeDtypeStruct((N2, HC), jnp.float32)],
        mesh=mesh,
        scratch_types=[
            pltpu.VMEM((_CW,), jnp.int32),
            pltpu.VMEM((4016,), jnp.int32),
            pltpu.VMEM((4016,), jnp.int32),
            pltpu.VMEM((_CG, 16), jnp.float32),
            pltpu.VMEM((_CG,), jnp.int32),
            pltpu.VMEM((_CG, HC), jnp.float32),
            pltpu.VMEM((_CCH, HC), jnp.float32),
            pltpu.VMEM((_CCH, 16), jnp.float32),
            pltpu.VMEM((_CCH, 16), jnp.float32),
            pltpu.VMEM((HC,), jnp.float32),
            pltpu.SemaphoreType.DMA,
        ])
    return f(dst, rec, h, dpart, bg)[0]


_DW = 16000
_DG = 64
_DCH = 320


def _d_kernel(dst_h, src_h, y1_h, z_h, dpart_h, x2_h,
              dstw, mids, mdst, srcg, ybuf, acc, zloc, dloc, dloc2, sem):
    cid = lax.axis_index("c")
    sid = lax.axis_index("s")
    wid = sid * NC + cid
    lane = lax.iota(jnp.int32, 16)
    lo = wid * _DCH

    @pl.loop(0, 1001)
    def _zm(k):
        mids[pl.ds(k * 16, 16)] = jnp.zeros((16,), jnp.int32)

    @pl.loop(0, _DCH)
    def _za(i):
        for v in range(CC // 16):
            acc[i, pl.ds(v * 16, 16)] = jnp.zeros((16,), jnp.float32)

    @pl.loop(0, EE // _DW)
    def _win(w):
        wb = w * _DW
        pltpu.sync_copy(dst_h.at[pl.ds(wb, _DW)], dstw)

        def _scan(k, cnt):
            dl = dstw[pl.ds(k * 16, 16)] - lo
            msk = (dl >= 0) & (dl < _DCH)
            ids = lane + (wb + k * 16)
            plsc.store_compressed(mids.at[pl.ds(cnt, 16)], ids, mask=msk)
            plsc.store_compressed(mdst.at[pl.ds(cnt, 16)], dl, mask=msk)
            return cnt + jnp.sum(msk.astype(jnp.int32))

        cnt = lax.fori_loop(0, _DW // 16, _scan, 0)
        nb = (cnt + _DG - 1) // _DG

        @pl.loop(0, nb)
        def _bat(b):
            mb = b * _DG
            pltpu.async_copy(src_h.at[mids.at[pl.ds(mb, _DG)]], srcg, sem).wait()
            pltpu.async_copy(y1_h.at[srcg], ybuf, sem).wait()

            @pl.loop(0, _DG)
            def _acc(g):
                @pl.when(mb + g < cnt)
                def _():
                    row = mdst[mb + g]
                    for v in range(CC // 16):
                        plsc.addupdate(acc.at[row, pl.ds(v * 16, 16)],
                                       ybuf[g, pl.ds(v * 16, 16)])

    pltpu.sync_copy(dpart_h.at[0, pl.ds(lo, _DCH)], dloc)
    pltpu.sync_copy(dpart_h.at[1, pl.ds(lo, _DCH)], dloc2)
    pltpu.sync_copy(z_h.at[pl.ds(lo, _DCH)], zloc)

    @pl.loop(0, _DCH)
    def _fin(i):
        dloc[i] = dloc[i] + dloc2[i]
        degb = jnp.maximum(
            plsc.load_gather(dloc, [jnp.full((16,), i, jnp.int32),
                                    jnp.full((16,), 8, jnp.int32)]), 1.0)
        for v in range(CC // 16):
            acc[i, pl.ds(v * 16, 16)] = jnp.maximum(
                acc[i, pl.ds(v * 16, 16)] / degb + zloc[i, pl.ds(v * 16, 16)], 0.0)

    pltpu.sync_copy(acc, x2_h.at[pl.ds(lo, _DCH)])


def _d(dst, src, y1, z, dpart):
    mesh = plsc.VectorSubcoreMesh(core_axis_name="c", subcore_axis_name="s",
                                  num_cores=NC, num_subcores=NS)
    f = pl.kernel(
        _d_kernel,
        out_type=[jax.ShapeDtypeStruct((N2, CC), jnp.float32)],
        mesh=mesh,
        scratch_types=[
            pltpu.VMEM((_DW,), jnp.int32),
            pltpu.VMEM((16016,), jnp.int32),
            pltpu.VMEM((16016,), jnp.int32),
            pltpu.VMEM((_DG,), jnp.int32),
            pltpu.VMEM((_DG, CC), jnp.float32),
            pltpu.VMEM((_DCH, CC), jnp.float32),
            pltpu.VMEM((_DCH, CC), jnp.float32),
            pltpu.VMEM((_DCH, 16), jnp.float32),
            pltpu.VMEM((_DCH, 16), jnp.float32),
            pltpu.SemaphoreType.DMA,
        ])
    return f(dst, src, y1, z, dpart)[0]


def kernel(x, edge_index, edge_weight, W_gat, att_src, att_dst, att_edge,
           W_edge, b_gat, W_sage_l, W_sage_r, b_sage, W_lin1, b_lin1,
           W_lin2, b_lin2):
    src = edge_index[0]
    dst = edge_index[1]

    PS = jnp.zeros((HC, 16), jnp.float32)
    PD = jnp.zeros((HC, 16), jnp.float32)
    for hh in range(HH):
        PS = PS.at[hh * CC:(hh + 1) * CC, hh].set(att_src[hh])
        PD = PD.at[hh * CC:(hh + 1) * CC, hh].set(att_dst[hh])

    cvec = (W_edge.reshape(HH, CC) * att_edge).sum(-1)
    crep = jnp.concatenate([cvec, jnp.zeros((8,), jnp.float32)])

    h, As, Ad = _tc1(x, W_gat, PS, PD)
    rec, dpart = _b1(src, dst, edge_weight, As, Ad, crep)
    x1 = _c(dst, rec, h, dpart, b_gat)
    y1, z = _tc2(x1, W_sage_l, W_sage_r, b_sage.reshape(1, CC))
    x2 = _d(dst, src, y1, z, dpart)
    out = _tc3(x2, W_lin1, b_lin1.reshape(1, 32), W_lin2, b_lin2.reshape(1, 1))
    return out[:NN]


# speedup vs baseline: 1.6326x; 1.6326x over previous
import jax
import jax.numpy as jnp
from jax import lax
from jax.experimental import pallas as pl
from jax.experimental.pallas import tpu as pltpu
from jax.experimental.pallas import tpu_sc as plsc

NN = 10000
EE = 320000
HH = 8
CC = 64
HC = 512
N2 = 10240
NC = 2
NS = 16
NW = NC * NS


def _tc1_body(x_ref, wg_ref, p_ref, h_ref, at_ref):
    xb = x_ref[...]
    hb = jnp.dot(xb, wg_ref[...], preferred_element_type=jnp.float32)
    h_ref[...] = hb
    at_ref[...] = jnp.dot(hb, p_ref[...], preferred_element_type=jnp.float32)


def _tc1(x, W_gat, P):
    R = 1000
    return pl.pallas_call(
        _tc1_body,
        grid=(NN // R,),
        in_specs=[pl.BlockSpec((R, 128), lambda i: (i, 0)),
                  pl.BlockSpec((128, HC), lambda i: (0, 0)),
                  pl.BlockSpec((HC, 128), lambda i: (0, 0))],
        out_specs=[pl.BlockSpec((R, HC), lambda i: (i, 0)),
                   pl.BlockSpec((R, 128), lambda i: (i, 0))],
        out_shape=[jax.ShapeDtypeStruct((NN, HC), jnp.float32),
                   jax.ShapeDtypeStruct((NN, 128), jnp.float32)],
    )(x, W_gat, P)


def _tc2_body(x1_ref, wl_ref, wr_ref, bs_ref, y1_ref, z_ref):
    xb = x1_ref[...]
    y1_ref[...] = jnp.dot(xb, wl_ref[...], preferred_element_type=jnp.float32)
    z_ref[...] = jnp.dot(xb, wr_ref[...], preferred_element_type=jnp.float32) + bs_ref[...]


def _tc2(x1, Wl, Wr, bs):
    R = 512
    return pl.pallas_call(
        _tc2_body,
        grid=(N2 // R,),
        in_specs=[pl.BlockSpec((R, HC), lambda i: (i, 0)),
                  pl.BlockSpec((HC, 128), lambda i: (0, 0)),
                  pl.BlockSpec((HC, CC), lambda i: (0, 0)),
                  pl.BlockSpec((1, CC), lambda i: (0, 0))],
        out_specs=[pl.BlockSpec((R, 128), lambda i: (i, 0)),
                   pl.BlockSpec((R, CC), lambda i: (i, 0))],
        out_shape=[jax.ShapeDtypeStruct((N2, 128), jnp.float32),
                   jax.ShapeDtypeStruct((N2, CC), jnp.float32)],
    )(x1, Wl, Wr, bs)


def _tc3_body(x2_ref, w1_ref, b1_ref, w2_ref, b2_ref, o_ref):
    xb = x2_ref[...]
    t = jnp.maximum(
        jnp.dot(xb, w1_ref[...], preferred_element_type=jnp.float32) + b1_ref[...], 0.0)
    o_ref[...] = jnp.dot(t, w2_ref[...], preferred_element_type=jnp.float32) + b2_ref[...]


def _tc3(x2, W1, b1, W2, b2):
    R = 512
    return pl.pallas_call(
        _tc3_body,
        grid=(N2 // R,),
        in_specs=[pl.BlockSpec((R, CC), lambda i: (i, 0)),
                  pl.BlockSpec((CC, 32), lambda i: (0, 0)),
                  pl.BlockSpec((1, 32), lambda i: (0, 0)),
                  pl.BlockSpec((32, 1), lambda i: (0, 0)),
                  pl.BlockSpec((1, 1), lambda i: (0, 0))],
        out_specs=pl.BlockSpec((R, 1), lambda i: (i, 0)),
        out_shape=jax.ShapeDtypeStruct((N2, 1), jnp.float32),
    )(x2, W1, b1, W2, b2)


def _tcsum_body(dp_ref, o_ref):
    s = dp_ref[0] + dp_ref[1]
    o_ref[...] = s[:, :16]


def _tcsum(dpart):
    R = 512
    return pl.pallas_call(
        _tcsum_body,
        grid=(N2 // R,),
        in_specs=[pl.BlockSpec((NC, R, 128), lambda i: (0, i, 0))],
        out_specs=pl.BlockSpec((R, 16), lambda i: (i, 0)),
        out_shape=jax.ShapeDtypeStruct((N2, 16), jnp.float32),
    )(dpart)


def _tree_sum(st, v):
    st[pl.ds(0, 16)] = v
    for sh in (8, 4, 2, 1):
        st[pl.ds(0, 16)] = st[pl.ds(0, 16)] + st[pl.ds(sh, 16)]
    return st[pl.ds(0, 16)][0]


def _scan_compact(dstw, srcw, eww, srcm, dstm, ewm, st, stf, lane, p2,
                  nvec, lo, ch, with_ew):
    one = jnp.ones((16,), jnp.int32)
    zero = jnp.zeros((16,), jnp.int32)

    def wbody(k, cnt):
        dv = dstw[pl.ds(k * 16, 16)]
        dl = dv - lo
        m = (dl >= 0) & (dl < ch)
        mi = jnp.where(m, one, zero)
        packed = _tree_sum(st, mi * p2 + mi * (1 << 20))
        nm = jnp.right_shift(packed, 20)
        bits0 = jnp.bitwise_and(packed, (1 << 20) - 1)

        def inner(j, carry):
            bits, c = carry
            lsb1 = jnp.bitwise_and(bits, -bits) - 1
            bl = jnp.bitwise_and(
                jnp.right_shift(jnp.full((16,), lsb1, jnp.int32), lane), 1)
            i = _tree_sum(stf, bl)
            sval = srcw[pl.ds(k * 16 + i, 16)][0]
            dval = dstw[pl.ds(k * 16 + i, 16)][0]
            srcm[pl.ds(c, 16)] = jnp.full((16,), sval, jnp.int32)
            dstm[pl.ds(c, 16)] = jnp.full((16,), dval, jnp.int32)
            if with_ew:
                ewval = eww[pl.ds(k * 16 + i, 16)][0]
                ewm[pl.ds(c, 16)] = jnp.full((16,), ewval, jnp.float32)
            return (jnp.bitwise_and(bits, bits - 1), c + 1)

        carry = lax.fori_loop(0, nm, inner, (bits0, cnt))
        return carry[1]

    return lax.fori_loop(0, nvec, wbody, 0)


_B1B = 80
_B1NB = EE // NW // _B1B


def _b1_kernel(src_h, dst_h, ew_h, at_h, crep_h, dpart_h,
               srcb, dstb, ewb, rowS, rowD, stg, crepv, shd, semS, semD):
    cid = lax.axis_index("c")
    sid = lax.axis_index("s")
    wid = sid * NC + cid
    lane = lax.iota(jnp.int32, 16)
    pltpu.sync_copy(crep_h, crepv)
    crep = crepv[...]
    one = jnp.ones((16,), jnp.float32)
    zero = jnp.zeros((16,), jnp.float32)

    @pl.loop(0, _B1B)
    def _zs(i):
        @pl.loop(0, 8)
        def _zl(k):
            stg[i, pl.ds(k * 16, 16)] = zero
    @pl.loop(0, 8)
    def _zc(q):
        pltpu.sync_copy(stg, shd.at[pl.ds(sid * 640 + q * _B1B, _B1B)])
    plsc.subcore_barrier()

    ebase = wid * (EE // NW)

    @pl.loop(0, _B1NB)
    def _batch(b):
        base = ebase + b * _B1B
        pltpu.sync_copy(src_h.at[pl.ds(base, _B1B)], srcb.at[pl.ds(0, _B1B)])
        pltpu.sync_copy(dst_h.at[pl.ds(base, _B1B)], dstb.at[pl.ds(0, _B1B)])
        pltpu.sync_copy(ew_h.at[pl.ds(base, _B1B)], ewb.at[pl.ds(0, _B1B)])
        cpS = pltpu.async_copy(at_h.at[srcb.at[pl.ds(0, _B1B)]], rowS, semS)
        cpD = pltpu.async_copy(at_h.at[dstb.at[pl.ds(0, _B1B)]], rowD, semD)
        cpS.wait()
        cpD.wait()

        @pl.loop(0, _B1B)
        def _e(i):
            rs = rowS[i, pl.ds(0, 16)]
            rd = rowD[i, pl.ds(8, 16)]
            ewv = jnp.full((16,), ewb[pl.ds(i, 16)][0], jnp.float32)
            lg = rs + rd + ewv * crep
            ex = jnp.exp(jnp.maximum(lg, 0.2 * lg))
            stg[i, pl.ds(0, 16)] = jnp.where(
                lane < 8, ex, jnp.where(lane == 8, one, zero))

        pltpu.sync_copy(stg, shd.at[dstb.at[pl.ds(0, _B1B)]], add=True)

    plsc.subcore_barrier()
    pltpu.sync_copy(shd.at[pl.ds(sid * 640, 640)],
                    dpart_h.at[cid, pl.ds(sid * 640, 640)])


def _b1(src, dst, ew, AT, crep):
    mesh = plsc.VectorSubcoreMesh(core_axis_name="c", subcore_axis_name="s",
                                  num_cores=NC, num_subcores=NS)
    f = pl.kernel(
        _b1_kernel,
        out_type=[jax.ShapeDtypeStruct((NC, N2, 128), jnp.float32)],
        mesh=mesh,
        scratch_types=[
            pltpu.VMEM((_B1B + 16,), jnp.int32),
            pltpu.VMEM((_B1B + 16,), jnp.int32),
            pltpu.VMEM((_B1B + 16,), jnp.float32),
            pltpu.VMEM((_B1B, 128), jnp.float32),
            pltpu.VMEM((_B1B, 128), jnp.float32),
            pltpu.VMEM((_B1B, 128), jnp.float32),
            pltpu.VMEM((16,), jnp.float32),
            pltpu.VMEM_SHARED((N2, 128), jnp.float32),
            pltpu.SemaphoreType.DMA,
            pltpu.SemaphoreType.DMA,
        ])
    return f(src, dst, ew, AT, crep)[0]


_CW = 4000
_CG = 16
_CCH = 128


def _c_kernel(src_h, dst_h, ew_h, at_h, crep_h, h_hbm, dp_h, bg_h,
              x1_h,
              dstw, srcw, eww, srcm, dstm, ewm, atS, atD, hbuf, acc,
              dloc, bgv, crepv, st, stf, astash, dstash, sem):
    cid = lax.axis_index("c")
    sid = lax.axis_index("s")
    wid = sid * NC + cid
    lane = lax.iota(jnp.int32, 16)
    p2 = jnp.left_shift(jnp.ones((16,), jnp.int32), lane)
    zero = jnp.zeros((16,), jnp.float32)
    izero = jnp.zeros((16,), jnp.int32)
    pltpu.sync_copy(bg_h, bgv)
    pltpu.sync_copy(crep_h, crepv)
    crep = crepv[...]
    st[pl.ds(16, 16)] = izero
    stf[pl.ds(16, 16)] = izero

    @pl.loop(0, (_CW + 16) // 16)
    def _zm(k):
        srcm[pl.ds(k * 16, 16)] = izero
        dstm[pl.ds(k * 16, 16)] = izero

    @pl.loop(0, 3)
    def _round(r):
        chunk = r * NW + wid

        @pl.when(chunk < N2 // _CCH)
        def _do():
            lo = chunk * _CCH

            @pl.loop(0, _CCH)
            def _za(i):
                @pl.loop(0, HC // 16)
                def _zv(v):
                    acc[i, pl.ds(v * 16, 16)] = zero

            @pl.loop(0, EE // _CW)
            def _win(w):
                wb = w * _CW
                pltpu.sync_copy(dst_h.at[pl.ds(wb, _CW)],
                                dstw.at[pl.ds(0, _CW)])
                pltpu.sync_copy(src_h.at[pl.ds(wb, _CW)],
                                srcw.at[pl.ds(0, _CW)])
                pltpu.sync_copy(ew_h.at[pl.ds(wb, _CW)],
                                eww.at[pl.ds(0, _CW)])
                cnt = _scan_compact(dstw, srcw, eww, srcm, dstm, ewm, st, stf,
                                    lane, p2, _CW // 16, lo, _CCH, True)
                nb = (cnt + _CG - 1) // _CG

                @pl.loop(0, nb)
                def _bat(b):
                    mb = b * _CG
                    cp1 = pltpu.async_copy(
                        at_h.at[srcm.at[pl.ds(mb, _CG)]], atS, sem)
                    cp2 = pltpu.async_copy(
                        at_h.at[dstm.at[pl.ds(mb, _CG)]], atD, sem)
                    cp3 = pltpu.async_copy(
                        h_hbm.at[srcm.at[pl.ds(mb, _CG)]], hbuf, sem)
                    cp1.wait()
                    cp2.wait()
                    cp3.wait()

                    @pl.loop(0, _CG)
                    def _acc(g):
                        @pl.when(mb + g < cnt)
                        def _():
                            ewv = jnp.full((16,), ewm[pl.ds(mb + g, 16)][0],
                                           jnp.float32)
                            lg = atS[g, pl.ds(0, 16)] + atD[g, pl.ds(8, 16)] \
                                + ewv * crep
                            av = jnp.exp(jnp.maximum(lg, 0.2 * lg))
                            astash[pl.ds(0, 16)] = av
                            row = dstm[pl.ds(mb + g, 16)][0] - lo

                            @pl.loop(0, HH)
                            def _h(h8):
                                ab = jnp.full(
                                    (16,), astash[pl.ds(h8, 16)][0],
                                    jnp.float32)
                                for v in range(4):
                                    col = h8 * CC + v * 16
                                    hv = hbuf[g, pl.ds(col, 16)]
                                    plsc.addupdate(
                                        acc.at[row, pl.ds(col, 16)], hv * ab)

            pltpu.sync_copy(dp_h.at[pl.ds(lo * 16, _CCH * 16)], dloc)

            @pl.loop(0, _CCH)
            def _fin(i):
                dstash[pl.ds(0, 16)] = (dloc[pl.ds(i * 16, 16)]
                                        + jnp.full((16,), 1e-16, jnp.float32))

                @pl.loop(0, HH)
                def _h(h8):
                    den = jnp.full((16,), dstash[pl.ds(h8, 16)][0],
                                   jnp.float32)
                    for v in range(4):
                        col = h8 * CC + v * 16
                        acc[i, pl.ds(col, 16)] = jnp.maximum(
                            acc[i, pl.ds(col, 16)] / den
                            + bgv[pl.ds(col, 16)], 0.0)

            pltpu.sync_copy(acc, x1_h.at[pl.ds(lo, _CCH)])


def _c(src, dst, ew, AT, crep, h, dp, bg):
    mesh = plsc.VectorSubcoreMesh(core_axis_name="c", subcore_axis_name="s",
                                  num_cores=NC, num_subcores=NS)
    f = pl.kernel(
        _c_kernel,
        out_type=[jax.ShapeDtypeStruct((N2, HC), jnp.float32)],
        mesh=mesh,
        scratch_types=[
            pltpu.VMEM((_CW + 16,), jnp.int32),
            pltpu.VMEM((_CW + 16,), jnp.int32),
            pltpu.VMEM((_CW + 16,), jnp.float32),
            pltpu.VMEM((_CW + 16,), jnp.int32),
            pltpu.VMEM((_CW + 16,), jnp.int32),
            pltpu.VMEM((_CW + 16,), jnp.float32),
            pltpu.VMEM((_CG, 128), jnp.float32),
            pltpu.VMEM((_CG, 128), jnp.float32),
            pltpu.VMEM((_CG, HC), jnp.float32),
            pltpu.VMEM((_CCH, HC), jnp.float32),
            pltpu.VMEM((_CCH * 16,), jnp.float32),
            pltpu.VMEM((HC,), jnp.float32),
            pltpu.VMEM((16,), jnp.float32),
            pltpu.VMEM((32,), jnp.int32),
            pltpu.VMEM((32,), jnp.int32),
            pltpu.VMEM((32,), jnp.float32),
            pltpu.VMEM((32,), jnp.float32),
            pltpu.SemaphoreType.DMA,
        ])
    return f(src, dst, ew, AT, crep, h, dp, bg)[0]


_DW = 4000
_DG = 32
_DCH = 320


def _d_kernel(src_h, dst_h, y1_h, z_h, dp_h, x2_h,
              dstw, srcw, srcm, dstm, ybuf, acc, zloc, dloc,
              st, stf, dstash, sem):
    cid = lax.axis_index("c")
    sid = lax.axis_index("s")
    wid = sid * NC + cid
    lane = lax.iota(jnp.int32, 16)
    p2 = jnp.left_shift(jnp.ones((16,), jnp.int32), lane)
    zero = jnp.zeros((16,), jnp.float32)
    izero = jnp.zeros((16,), jnp.int32)
    lo = wid * _DCH
    st[pl.ds(16, 16)] = izero
    stf[pl.ds(16, 16)] = izero

    @pl.loop(0, (_DW + 16) // 16)
    def _zm(k):
        srcm[pl.ds(k * 16, 16)] = izero
        dstm[pl.ds(k * 16, 16)] = izero

    @pl.loop(0, _DCH * CC // 16)
    def _za(i):
        acc[pl.ds(i * 16, 16)] = zero

    @pl.loop(0, EE // _DW)
    def _win(w):
        wb = w * _DW
        pltpu.sync_copy(dst_h.at[pl.ds(wb, _DW)], dstw.at[pl.ds(0, _DW)])
        pltpu.sync_copy(src_h.at[pl.ds(wb, _DW)], srcw.at[pl.ds(0, _DW)])
        cnt = _scan_compact(dstw, srcw, None, srcm, dstm, None, st, stf,
                            lane, p2, _DW // 16, lo, _DCH, False)
        nb = (cnt + _DG - 1) // _DG

        @pl.loop(0, nb)
        def _bat(b):
            mb = b * _DG
            pltpu.async_copy(y1_h.at[srcm.at[pl.ds(mb, _DG)]], ybuf,
                             sem).wait()

            @pl.loop(0, _DG)
            def _acc(g):
                @pl.when(mb + g < cnt)
                def _():
                    row = dstm[pl.ds(mb + g, 16)][0] - lo
                    for v in range(CC // 16):
                        plsc.addupdate(
                            acc.at[pl.ds(row * CC + v * 16, 16)],
                            ybuf[g, pl.ds(v * 16, 16)])

    pltpu.sync_copy(dp_h.at[pl.ds(lo * 16, _DCH * 16)], dloc)
    pltpu.sync_copy(z_h.at[pl.ds(lo * CC, _DCH * CC)], zloc)
    ei = sid * 0 + 8

    @pl.loop(0, _DCH)
    def _fin(i):
        dstash[pl.ds(0, 16)] = dloc[pl.ds(i * 16, 16)]
        degb = jnp.maximum(
            jnp.full((16,), dstash[pl.ds(ei, 16)][0], jnp.float32), 1.0)
        for v in range(CC // 16):
            o = i * CC + v * 16
            acc[pl.ds(o, 16)] = jnp.maximum(
                acc[pl.ds(o, 16)] / degb + zloc[pl.ds(o, 16)], 0.0)

    pltpu.sync_copy(acc, x2_h.at[pl.ds(lo * CC, _DCH * CC)])


def _d(src, dst, y1, z1d, dp):
    mesh = plsc.VectorSubcoreMesh(core_axis_name="c", subcore_axis_name="s",
                                  num_cores=NC, num_subcores=NS)
    f = pl.kernel(
        _d_kernel,
        out_type=[jax.ShapeDtypeStruct((N2 * CC,), jnp.float32)],
        mesh=mesh,
        scratch_types=[
            pltpu.VMEM((_DW + 16,), jnp.int32),
            pltpu.VMEM((_DW + 16,), jnp.int32),
            pltpu.VMEM((_DW + 16,), jnp.int32),
            pltpu.VMEM((_DW + 16,), jnp.int32),
            pltpu.VMEM((_DG, 128), jnp.float32),
            pltpu.VMEM((_DCH * CC,), jnp.float32),
            pltpu.VMEM((_DCH * CC,), jnp.float32),
            pltpu.VMEM((_DCH * 16,), jnp.float32),
            pltpu.VMEM((32,), jnp.int32),
            pltpu.VMEM((32,), jnp.int32),
            pltpu.VMEM((32,), jnp.float32),
            pltpu.SemaphoreType.DMA,
        ])
    return f(src, dst, y1, z1d, dp)[0]


def kernel(x, edge_index, edge_weight, W_gat, att_src, att_dst, att_edge,
           W_edge, b_gat, W_sage_l, W_sage_r, b_sage, W_lin1, b_lin1,
           W_lin2, b_lin2):
    src = edge_index[0]
    dst = edge_index[1]

    P = jnp.zeros((HC, 128), jnp.float32)
    for hh in range(HH):
        P = P.at[hh * CC:(hh + 1) * CC, hh].set(att_src[hh])
        P = P.at[hh * CC:(hh + 1) * CC, 8 + hh].set(att_dst[hh])

    cvec = (W_edge.reshape(HH, CC) * att_edge).sum(-1)
    crep = jnp.concatenate([cvec, jnp.zeros((8,), jnp.float32)])

    h, AT = _tc1(x, W_gat, P)
    dpart = _b1(src, dst, edge_weight, AT, crep)
    dp1d = _tcsum(dpart).reshape(-1)
    x1 = _c(src, dst, edge_weight, AT, crep, h, dp1d, b_gat)
    Wl = jnp.pad(W_sage_l, ((0, 0), (0, 128 - CC)))
    y1, z = _tc2(x1, Wl, W_sage_r, b_sage.reshape(1, CC))
    x2 = _d(src, dst, y1, z.reshape(-1), dp1d)
    out = _tc3(x2.reshape(N2, CC), W_lin1, b_lin1.reshape(1, 32),
               W_lin2, b_lin2.reshape(1, 1))
    return out[:NN]


# speedup vs baseline: 1.7346x; 1.0625x over previous
import jax
import jax.numpy as jnp
from jax import lax
from jax.experimental import pallas as pl
from jax.experimental.pallas import tpu as pltpu
from jax.experimental.pallas import tpu_sc as plsc

NN = 10000
EE = 320000
HH = 8
CC = 64
HC = 512
N2 = 10240
NC = 2
NS = 16
NW = NC * NS


def _tc1_body(x_ref, wg_ref, p_ref, h_ref, at_ref):
    xb = x_ref[...]
    hb = jnp.dot(xb, wg_ref[...], preferred_element_type=jnp.float32)
    h_ref[...] = hb
    at_ref[...] = jnp.dot(hb, p_ref[...], preferred_element_type=jnp.float32)


def _tc1(x, W_gat, P):
    R = 1000
    return pl.pallas_call(
        _tc1_body,
        grid=(NN // R,),
        in_specs=[pl.BlockSpec((R, 128), lambda i: (i, 0)),
                  pl.BlockSpec((128, HC), lambda i: (0, 0)),
                  pl.BlockSpec((HC, 128), lambda i: (0, 0))],
        out_specs=[pl.BlockSpec((R, HC), lambda i: (i, 0)),
                   pl.BlockSpec((R, 128), lambda i: (i, 0))],
        out_shape=[jax.ShapeDtypeStruct((NN, HC), jnp.float32),
                   jax.ShapeDtypeStruct((NN, 128), jnp.float32)],
    )(x, W_gat, P)


def _tc2_body(x1_ref, wl_ref, wr_ref, bs_ref, y1_ref, z_ref):
    xb = x1_ref[...]
    y1_ref[...] = jnp.dot(xb, wl_ref[...], preferred_element_type=jnp.float32)
    z_ref[...] = jnp.dot(xb, wr_ref[...], preferred_element_type=jnp.float32) + bs_ref[...]


def _tc2(x1, Wl, Wr, bs):
    R = 512
    return pl.pallas_call(
        _tc2_body,
        grid=(N2 // R,),
        in_specs=[pl.BlockSpec((R, HC), lambda i: (i, 0)),
                  pl.BlockSpec((HC, 128), lambda i: (0, 0)),
                  pl.BlockSpec((HC, CC), lambda i: (0, 0)),
                  pl.BlockSpec((1, CC), lambda i: (0, 0))],
        out_specs=[pl.BlockSpec((R, 128), lambda i: (i, 0)),
                   pl.BlockSpec((R, CC), lambda i: (i, 0))],
        out_shape=[jax.ShapeDtypeStruct((N2, 128), jnp.float32),
                   jax.ShapeDtypeStruct((N2, CC), jnp.float32)],
    )(x1, Wl, Wr, bs)


def _tc3_body(x2_ref, w1_ref, b1_ref, w2_ref, b2_ref, o_ref):
    xb = x2_ref[...]
    t = jnp.maximum(
        jnp.dot(xb, w1_ref[...], preferred_element_type=jnp.float32) + b1_ref[...], 0.0)
    o_ref[...] = jnp.dot(t, w2_ref[...], preferred_element_type=jnp.float32) + b2_ref[...]


def _tc3(x2, W1, b1, W2, b2):
    R = 512
    return pl.pallas_call(
        _tc3_body,
        grid=(N2 // R,),
        in_specs=[pl.BlockSpec((R, CC), lambda i: (i, 0)),
                  pl.BlockSpec((CC, 32), lambda i: (0, 0)),
                  pl.BlockSpec((1, 32), lambda i: (0, 0)),
                  pl.BlockSpec((32, 1), lambda i: (0, 0)),
                  pl.BlockSpec((1, 1), lambda i: (0, 0))],
        out_specs=pl.BlockSpec((R, 1), lambda i: (i, 0)),
        out_shape=jax.ShapeDtypeStruct((N2, 1), jnp.float32),
    )(x2, W1, b1, W2, b2)


def _tcsum_body(dp_ref, o_ref):
    s = dp_ref[0] + dp_ref[1]
    o_ref[...] = s[:, :16]


def _tcsum(dpart):
    R = 512
    return pl.pallas_call(
        _tcsum_body,
        grid=(N2 // R,),
        in_specs=[pl.BlockSpec((NC, R, 128), lambda i: (0, i, 0))],
        out_specs=pl.BlockSpec((R, 16), lambda i: (i, 0)),
        out_shape=jax.ShapeDtypeStruct((N2, 16), jnp.float32),
    )(dpart)


def _tree_sum(st, v):
    st[pl.ds(0, 16)] = v
    for sh in (8, 4, 2, 1):
        st[pl.ds(0, 16)] = st[pl.ds(0, 16)] + st[pl.ds(sh, 16)]
    return st[pl.ds(0, 16)][0]


def _scan_compact(dstw, srcw, eww, srcm, dstm, ewm, st, stf, lane, p2,
                  nvec, lo, ch, with_ew):
    one = jnp.ones((16,), jnp.int32)
    zero = jnp.zeros((16,), jnp.int32)

    def wbody(k, cnt):
        dv = dstw[pl.ds(k * 16, 16)]
        dl = dv - lo
        m = (dl >= 0) & (dl < ch)
        mi = jnp.where(m, one, zero)
        packed = _tree_sum(st, mi * p2 + mi * (1 << 20))
        nm = jnp.right_shift(packed, 20)
        bits0 = jnp.bitwise_and(packed, (1 << 20) - 1)

        def inner(j, carry):
            bits, c = carry
            lsb1 = jnp.bitwise_and(bits, -bits) - 1
            bl = jnp.bitwise_and(
                jnp.right_shift(jnp.full((16,), lsb1, jnp.int32), lane), 1)
            i = _tree_sum(stf, bl)
            sval = srcw[pl.ds(k * 16 + i, 16)][0]
            dval = dstw[pl.ds(k * 16 + i, 16)][0]
            srcm[pl.ds(c, 16)] = jnp.full((16,), sval, jnp.int32)
            dstm[pl.ds(c, 16)] = jnp.full((16,), dval, jnp.int32)
            if with_ew:
                ewval = eww[pl.ds(k * 16 + i, 16)][0]
                ewm[pl.ds(c, 16)] = jnp.full((16,), ewval, jnp.float32)
            return (jnp.bitwise_and(bits, bits - 1), c + 1)

        carry = lax.fori_loop(0, nm, inner, (bits0, cnt))
        return carry[1]

    return lax.fori_loop(0, nvec, wbody, 0)


_B1B = 80
_B1NB = EE // NW // _B1B


def _b1_kernel(src_h, dst_h, ew_h, at_h, crep_h, dpart_h,
               srcb, dstb, ewb, rowS, rowD, stg, crepv, shd, semS, semD):
    cid = lax.axis_index("c")
    sid = lax.axis_index("s")
    wid = sid * NC + cid
    lane = lax.iota(jnp.int32, 16)
    pltpu.sync_copy(crep_h, crepv)
    crep = crepv[...]
    one = jnp.ones((16,), jnp.float32)
    zero = jnp.zeros((16,), jnp.float32)

    @pl.loop(0, _B1B)
    def _zs(i):
        @pl.loop(0, 8)
        def _zl(k):
            stg[i, pl.ds(k * 16, 16)] = zero
    @pl.loop(0, 8)
    def _zc(q):
        pltpu.sync_copy(stg, shd.at[pl.ds(sid * 640 + q * _B1B, _B1B)])
    plsc.subcore_barrier()

    ebase = wid * (EE // NW)

    @pl.loop(0, _B1NB)
    def _batch(b):
        base = ebase + b * _B1B
        w1 = pltpu.async_copy(src_h.at[pl.ds(base, _B1B)],
                              srcb.at[pl.ds(0, _B1B)], semS)
        w2 = pltpu.async_copy(dst_h.at[pl.ds(base, _B1B)],
                              dstb.at[pl.ds(0, _B1B)], semD)
        w3 = pltpu.async_copy(ew_h.at[pl.ds(base, _B1B)],
                              ewb.at[pl.ds(0, _B1B)], semS)
        w1.wait()
        w2.wait()
        w3.wait()
        cpS = pltpu.async_copy(at_h.at[srcb.at[pl.ds(0, _B1B)]], rowS, semS)
        cpD = pltpu.async_copy(at_h.at[dstb.at[pl.ds(0, _B1B)]], rowD, semD)
        cpS.wait()
        cpD.wait()

        @pl.loop(0, _B1B)
        def _e(i):
            rs = rowS[i, pl.ds(0, 16)]
            rd = rowD[i, pl.ds(8, 16)]
            ewv = jnp.full((16,), ewb[pl.ds(i, 16)][0], jnp.float32)
            lg = rs + rd + ewv * crep
            ex = jnp.exp(jnp.maximum(lg, 0.2 * lg))
            stg[i, pl.ds(0, 16)] = jnp.where(
                lane < 8, ex, jnp.where(lane == 8, one, zero))

        pltpu.sync_copy(stg, shd.at[dstb.at[pl.ds(0, _B1B)]], add=True)

    plsc.subcore_barrier()
    pltpu.sync_copy(shd.at[pl.ds(sid * 640, 640)],
                    dpart_h.at[cid, pl.ds(sid * 640, 640)])


def _b1(src, dst, ew, AT, crep):
    mesh = plsc.VectorSubcoreMesh(core_axis_name="c", subcore_axis_name="s",
                                  num_cores=NC, num_subcores=NS)
    f = pl.kernel(
        _b1_kernel,
        out_type=[jax.ShapeDtypeStruct((NC, N2, 128), jnp.float32)],
        mesh=mesh,
        scratch_types=[
            pltpu.VMEM((_B1B + 16,), jnp.int32),
            pltpu.VMEM((_B1B + 16,), jnp.int32),
            pltpu.VMEM((_B1B + 16,), jnp.float32),
            pltpu.VMEM((_B1B, 128), jnp.float32),
            pltpu.VMEM((_B1B, 128), jnp.float32),
            pltpu.VMEM((_B1B, 128), jnp.float32),
            pltpu.VMEM((16,), jnp.float32),
            pltpu.VMEM_SHARED((N2, 128), jnp.float32),
            pltpu.SemaphoreType.DMA,
            pltpu.SemaphoreType.DMA,
        ])
    return f(src, dst, ew, AT, crep)[0]


_CW = 4000
_CG = 32
_CCH = 128


def _c_kernel(src_h, dst_h, ew_h, at_h, crep_h, h_hbm, dp_h, bg_h,
              x1_h,
              dstw, srcw, eww, srcm, dstm, ewm, atS, atD, hbuf, acc,
              dloc, bgv, crepv, st, stf, astash, dstash, sem):
    cid = lax.axis_index("c")
    sid = lax.axis_index("s")
    wid = sid * NC + cid
    lane = lax.iota(jnp.int32, 16)
    p2 = jnp.left_shift(jnp.ones((16,), jnp.int32), lane)
    zero = jnp.zeros((16,), jnp.float32)
    izero = jnp.zeros((16,), jnp.int32)
    pltpu.sync_copy(bg_h, bgv)
    pltpu.sync_copy(crep_h, crepv)
    crep = crepv[...]
    st[pl.ds(16, 16)] = izero
    stf[pl.ds(16, 16)] = izero

    @pl.loop(0, (_CW + 16) // 16)
    def _zm(k):
        srcm[pl.ds(k * 16, 16)] = izero
        dstm[pl.ds(k * 16, 16)] = izero

    @pl.loop(0, 3)
    def _round(r):
        chunk = r * NW + wid

        @pl.when(chunk < N2 // _CCH)
        def _do():
            lo = chunk * _CCH

            @pl.loop(0, _CCH)
            def _za(i):
                @pl.loop(0, HC // 16)
                def _zv(v):
                    acc[i, pl.ds(v * 16, 16)] = zero

            @pl.loop(0, EE // _CW)
            def _win(w):
                wb = w * _CW
                w1 = pltpu.async_copy(dst_h.at[pl.ds(wb, _CW)],
                                      dstw.at[pl.ds(0, _CW)], sem)
                w2 = pltpu.async_copy(src_h.at[pl.ds(wb, _CW)],
                                      srcw.at[pl.ds(0, _CW)], sem)
                w3 = pltpu.async_copy(ew_h.at[pl.ds(wb, _CW)],
                                      eww.at[pl.ds(0, _CW)], sem)
                w1.wait()
                w2.wait()
                w3.wait()
                cnt = _scan_compact(dstw, srcw, eww, srcm, dstm, ewm, st, stf,
                                    lane, p2, _CW // 16, lo, _CCH, True)
                nb = (cnt + _CG - 1) // _CG

                @pl.loop(0, nb)
                def _bat(b):
                    mb = b * _CG
                    cp1 = pltpu.async_copy(
                        at_h.at[srcm.at[pl.ds(mb, _CG)]], atS, sem)
                    cp2 = pltpu.async_copy(
                        at_h.at[dstm.at[pl.ds(mb, _CG)]], atD, sem)
                    cp3 = pltpu.async_copy(
                        h_hbm.at[srcm.at[pl.ds(mb, _CG)]], hbuf, sem)
                    cp1.wait()
                    cp2.wait()
                    cp3.wait()

                    @pl.loop(0, _CG)
                    def _acc(g):
                        @pl.when(mb + g < cnt)
                        def _():
                            ewv = jnp.full((16,), ewm[pl.ds(mb + g, 16)][0],
                                           jnp.float32)
                            lg = atS[g, pl.ds(0, 16)] + atD[g, pl.ds(8, 16)] \
                                + ewv * crep
                            av = jnp.exp(jnp.maximum(lg, 0.2 * lg))
                            astash[pl.ds(0, 16)] = av
                            row = dstm[pl.ds(mb + g, 16)][0] - lo

                            @pl.loop(0, HH)
                            def _h(h8):
                                ab = jnp.full(
                                    (16,), astash[pl.ds(h8, 16)][0],
                                    jnp.float32)
                                for v in range(4):
                                    col = h8 * CC + v * 16
                                    hv = hbuf[g, pl.ds(col, 16)]
                                    plsc.addupdate(
                                        acc.at[row, pl.ds(col, 16)], hv * ab)

            pltpu.sync_copy(dp_h.at[pl.ds(lo * 16, _CCH * 16)], dloc)

            @pl.loop(0, _CCH)
            def _fin(i):
                dstash[pl.ds(0, 16)] = (dloc[pl.ds(i * 16, 16)]
                                        + jnp.full((16,), 1e-16, jnp.float32))

                @pl.loop(0, HH)
                def _h(h8):
                    den = jnp.full((16,), dstash[pl.ds(h8, 16)][0],
                                   jnp.float32)
                    for v in range(4):
                        col = h8 * CC + v * 16
                        acc[i, pl.ds(col, 16)] = jnp.maximum(
                            acc[i, pl.ds(col, 16)] / den
                            + bgv[pl.ds(col, 16)], 0.0)

            pltpu.sync_copy(acc, x1_h.at[pl.ds(lo, _CCH)])


def _c(src, dst, ew, AT, crep, h, dp, bg):
    mesh = plsc.VectorSubcoreMesh(core_axis_name="c", subcore_axis_name="s",
                                  num_cores=NC, num_subcores=NS)
    f = pl.kernel(
        _c_kernel,
        out_type=[jax.ShapeDtypeStruct((N2, HC), jnp.float32)],
        mesh=mesh,
        scratch_types=[
            pltpu.VMEM((_CW + 16,), jnp.int32),
            pltpu.VMEM((_CW + 16,), jnp.int32),
            pltpu.VMEM((_CW + 16,), jnp.float32),
            pltpu.VMEM((_CW + 16,), jnp.int32),
            pltpu.VMEM((_CW + 16,), jnp.int32),
            pltpu.VMEM((_CW + 16,), jnp.float32),
            pltpu.VMEM((_CG, 128), jnp.float32),
            pltpu.VMEM((_CG, 128), jnp.float32),
            pltpu.VMEM((_CG, HC), jnp.float32),
            pltpu.VMEM((_CCH, HC), jnp.float32),
            pltpu.VMEM((_CCH * 16,), jnp.float32),
            pltpu.VMEM((HC,), jnp.float32),
            pltpu.VMEM((16,), jnp.float32),
            pltpu.VMEM((32,), jnp.int32),
            pltpu.VMEM((32,), jnp.int32),
            pltpu.VMEM((32,), jnp.float32),
            pltpu.VMEM((32,), jnp.float32),
            pltpu.SemaphoreType.DMA,
        ])
    return f(src, dst, ew, AT, crep, h, dp, bg)[0]


_DW = 4000
_DG = 32
_DCH = 320


def _d_kernel(src_h, dst_h, y1_h, z_h, dp_h, x2_h,
              dstw, srcw, srcm, dstm, ybuf, acc, zloc, dloc,
              st, stf, dstash, sem):
    cid = lax.axis_index("c")
    sid = lax.axis_index("s")
    wid = sid * NC + cid
    lane = lax.iota(jnp.int32, 16)
    p2 = jnp.left_shift(jnp.ones((16,), jnp.int32), lane)
    zero = jnp.zeros((16,), jnp.float32)
    izero = jnp.zeros((16,), jnp.int32)
    lo = wid * _DCH
    st[pl.ds(16, 16)] = izero
    stf[pl.ds(16, 16)] = izero

    @pl.loop(0, (_DW + 16) // 16)
    def _zm(k):
        srcm[pl.ds(k * 16, 16)] = izero
        dstm[pl.ds(k * 16, 16)] = izero

    @pl.loop(0, _DCH * CC // 16)
    def _za(i):
        acc[pl.ds(i * 16, 16)] = zero

    @pl.loop(0, EE // _DW)
    def _win(w):
        wb = w * _DW
        w1 = pltpu.async_copy(dst_h.at[pl.ds(wb, _DW)],
                              dstw.at[pl.ds(0, _DW)], sem)
        w2 = pltpu.async_copy(src_h.at[pl.ds(wb, _DW)],
                              srcw.at[pl.ds(0, _DW)], sem)
        w1.wait()
        w2.wait()
        cnt = _scan_compact(dstw, srcw, None, srcm, dstm, None, st, stf,
                            lane, p2, _DW // 16, lo, _DCH, False)
        nb = (cnt + _DG - 1) // _DG

        @pl.loop(0, nb)
        def _bat(b):
            mb = b * _DG
            pltpu.async_copy(y1_h.at[srcm.at[pl.ds(mb, _DG)]], ybuf,
                             sem).wait()

            @pl.loop(0, _DG)
            def _acc(g):
                @pl.when(mb + g < cnt)
                def _():
                    row = dstm[pl.ds(mb + g, 16)][0] - lo
                    for v in range(CC // 16):
                        plsc.addupdate(
                            acc.at[pl.ds(row * CC + v * 16, 16)],
                            ybuf[g, pl.ds(v * 16, 16)])

    pltpu.sync_copy(dp_h.at[pl.ds(lo * 16, _DCH * 16)], dloc)
    pltpu.sync_copy(z_h.at[pl.ds(lo * CC, _DCH * CC)], zloc)
    ei = sid * 0 + 8

    @pl.loop(0, _DCH)
    def _fin(i):
        dstash[pl.ds(0, 16)] = dloc[pl.ds(i * 16, 16)]
        degb = jnp.maximum(
            jnp.full((16,), dstash[pl.ds(ei, 16)][0], jnp.float32), 1.0)
        for v in range(CC // 16):
            o = i * CC + v * 16
            acc[pl.ds(o, 16)] = jnp.maximum(
                acc[pl.ds(o, 16)] / degb + zloc[pl.ds(o, 16)], 0.0)

    pltpu.sync_copy(acc, x2_h.at[pl.ds(lo * CC, _DCH * CC)])


def _d(src, dst, y1, z1d, dp):
    mesh = plsc.VectorSubcoreMesh(core_axis_name="c", subcore_axis_name="s",
                                  num_cores=NC, num_subcores=NS)
    f = pl.kernel(
        _d_kernel,
        out_type=[jax.ShapeDtypeStruct((N2 * CC,), jnp.float32)],
        mesh=mesh,
        scratch_types=[
            pltpu.VMEM((_DW + 16,), jnp.int32),
            pltpu.VMEM((_DW + 16,), jnp.int32),
            pltpu.VMEM((_DW + 16,), jnp.int32),
            pltpu.VMEM((_DW + 16,), jnp.int32),
            pltpu.VMEM((_DG, 128), jnp.float32),
            pltpu.VMEM((_DCH * CC,), jnp.float32),
            pltpu.VMEM((_DCH * CC,), jnp.float32),
            pltpu.VMEM((_DCH * 16,), jnp.float32),
            pltpu.VMEM((32,), jnp.int32),
            pltpu.VMEM((32,), jnp.int32),
            pltpu.VMEM((32,), jnp.float32),
            pltpu.SemaphoreType.DMA,
        ])
    return f(src, dst, y1, z1d, dp)[0]


def kernel(x, edge_index, edge_weight, W_gat, att_src, att_dst, att_edge,
           W_edge, b_gat, W_sage_l, W_sage_r, b_sage, W_lin1, b_lin1,
           W_lin2, b_lin2):
    src = edge_index[0]
    dst = edge_index[1]

    P = jnp.zeros((HC, 128), jnp.float32)
    for hh in range(HH):
        P = P.at[hh * CC:(hh + 1) * CC, hh].set(att_src[hh])
        P = P.at[hh * CC:(hh + 1) * CC, 8 + hh].set(att_dst[hh])

    cvec = (W_edge.reshape(HH, CC) * att_edge).sum(-1)
    crep = jnp.concatenate([cvec, jnp.zeros((8,), jnp.float32)])

    h, AT = _tc1(x, W_gat, P)
    dpart = _b1(src, dst, edge_weight, AT, crep)
    dp1d = _tcsum(dpart).reshape(-1)
    x1 = _c(src, dst, edge_weight, AT, crep, h, dp1d, b_gat)
    Wl = jnp.pad(W_sage_l, ((0, 0), (0, 128 - CC)))
    y1, z = _tc2(x1, Wl, W_sage_r, b_sage.reshape(1, CC))
    x2 = _d(src, dst, y1, z.reshape(-1), dp1d)
    out = _tc3(x2.reshape(N2, CC), W_lin1, b_lin1.reshape(1, 32),
               W_lin2, b_lin2.reshape(1, 1))
    return out[:NN]
